# baseline (speedup 1.0000x reference)
import functools

import jax
import jax.numpy as jnp
from jax import lax
from jax.experimental import pallas as pl
from jax.experimental.pallas import tpu as pltpu
from jax.experimental.pallas import tpu_sc as plsc

N_ROWS = 16384
TABLE_ROWS = 1_000_000
BASIC = 16
LANES = 16
NFEAT = 15

NUM_CORES = 2
NUM_SUBCORES = 16
NUM_WORKERS = NUM_CORES * NUM_SUBCORES
TILES_SMALL = 244
BLOCK = 1024
NPAIR = 15
CAP = 1024
CAPC = CAP + LANES
GCHUNK = 128
NCHUNKS = CAP // GCHUNK
DCAP = 64


@jax.jit
def _scatter_table_t(ids, *planes):
    mesh = plsc.VectorSubcoreMesh(core_axis_name="core", subcore_axis_name="subcore")

    @functools.partial(
        pl.kernel,
        out_type=jax.ShapeDtypeStruct((BASIC, TABLE_ROWS), jnp.float32),
        mesh=mesh,
        compiler_params=pltpu.CompilerParams(needs_layout_passes=False,
                                             disable_bounds_checks=True),
        scratch_types=[
            pltpu.VMEM((N_ROWS,), jnp.int32),
            pltpu.VMEM((CAPC,), jnp.int32),
            pltpu.VMEM((CAPC,), jnp.int32),
            pltpu.VMEM((BASIC * CAPC,), jnp.float32),
            pltpu.VMEM((BASIC, BLOCK), jnp.float32),
            pltpu.VMEM((BASIC, BLOCK), jnp.float32),
            pltpu.VMEM((CAPC,), jnp.int32),
            pltpu.VMEM((CAPC,), jnp.int32),
            pltpu.VMEM((2 * DCAP + LANES,), jnp.int32),
            pltpu.SemaphoreType.DMA,
            pltpu.SemaphoreType.DMA,
            pltpu.SemaphoreType.DMA,
        ],
    )
    def run(ids_hbm, *rest):
        plane_hbm = rest[:NFEAT]
        out_hbm = rest[NFEAT]
        (ids_v, rows_l, ids_l, vals_v, stage_a, stage_b, subp_v, subid_v,
         dlist_v, sem_g, sem_a, sem_b) = rest[NFEAT + 1:]

        wid = lax.axis_index("subcore") * NUM_CORES + lax.axis_index("core")
        tile_lo = wid * TILES_SMALL + jnp.minimum(wid, 4)
        col_lo = pl.multiple_of(tile_lo * 128, 128)
        ntiles = jnp.where(wid < 4, TILES_SMALL + 1, TILES_SMALL)
        col_hi = col_lo + ntiles * 128
        mask_hi = jnp.where(wid == NUM_WORKERS - 1, TABLE_ROWS, col_hi)
        has_tail = (wid < 4) | (wid == NUM_WORKERS - 1)
        iota = lax.iota(jnp.int32, LANES)
        lane0 = iota == 0
        zrow = jnp.zeros((LANES,), jnp.float32)
        neg1 = jnp.full((LANES,), -1, jnp.int32)
        c17 = jnp.full((LANES,), 17.0, jnp.float32)

        pltpu.sync_copy(ids_hbm, ids_v)

        @pl.loop(0, CAPC // LANES)
        def _(i):
            rows_l[pl.ds(i * LANES, LANES)] = (iota + i * LANES) * 8
            ids_l[pl.ds(i * LANES, LANES)] = neg1
            vals_v[pl.ds((BASIC - 1) * CAPC + i * LANES, LANES)] = c17

        @pl.loop(0, BLOCK // 4)
        def _(i):
            for u in range(4):
                cc = jnp.full((LANES,), i * 4 + u, jnp.int32)
                plsc.store_scatter(stage_a, [iota, cc], zrow)
                plsc.store_scatter(stage_b, [iota, cc], zrow)

        def scan_body(b, cnt):
            idv = ids_v[pl.ds(b * LANES, LANES)]
            m = (idv >= col_lo) & (idv < mask_hi)
            s = jnp.sum(m.astype(jnp.int32))

            @pl.when(s > 0)
            def _():
                plsc.store_compressed(rows_l.at[pl.ds(cnt, LANES)],
                                      iota + b * LANES, mask=m)
                plsc.store_compressed(ids_l.at[pl.ds(cnt, LANES)], idv, mask=m)

            return jnp.minimum(cnt + s, CAP)

        cnt = lax.fori_loop(0, N_ROWS // LANES, scan_body, 0)

        for k in range(NCHUNKS):
            @pl.when(k * GCHUNK < cnt)
            def _():
                idx = rows_l.at[pl.ds(k * GCHUNK, GCHUNK)]
                copies = [
                    pltpu.async_copy(
                        plane_hbm[r].at[idx],
                        vals_v.at[pl.ds(r * CAPC + k * GCHUNK, GCHUNK)],
                        sem_g)
                    for r in range(NFEAT)
                ]
                for c in copies:
                    c.wait()

        def rezero(stage, dslot, dcnt):
            def few(_):
                def zb(q, _):
                    cc = dlist_v[pl.ds(dslot * DCAP + q, LANES)][0]
                    plsc.store_scatter(stage,
                                       [iota, jnp.full((LANES,), cc, jnp.int32)],
                                       zrow)
                    return 0
                lax.fori_loop(0, dcnt, zb, 0)
                return 0

            def full(_):
                def zb(c, _):
                    plsc.store_scatter(stage,
                                       [iota, jnp.full((LANES,), c, jnp.int32)],
                                       zrow)
                    return 0
                lax.fori_loop(0, BLOCK, zb, 0)
                return 0

            lax.cond(dcnt <= DCAP, few, full, 0)

        def fill_block(s, stage, dslot):
            base = col_lo + s * BLOCK

            def sb(i, sc):
                idv = ids_l[pl.ds(i * LANES, LANES)]
                m = (idv >= base) & (idv < base + BLOCK)
                s2 = jnp.sum(m.astype(jnp.int32))

                @pl.when(s2 > 0)
                def _():
                    plsc.store_compressed(subp_v.at[pl.ds(sc, LANES)],
                                          iota + i * LANES, mask=m)
                    plsc.store_compressed(subid_v.at[pl.ds(sc, LANES)], idv,
                                          mask=m)

                return sc + s2

            nb = lax.fori_loop(0, (cnt + LANES - 1) // LANES, sb, 0)

            def wb(q, d):
                sp = subp_v[pl.ds(q, LANES)][0]
                sid = subid_v[pl.ds(q, LANES)][0]
                cc = sid - base
                val = plsc.load_gather(vals_v,
                                       [iota * CAPC + jnp.full((LANES,), sp,
                                                               jnp.int32)])
                plsc.store_scatter(stage,
                                   [iota, jnp.full((LANES,), cc, jnp.int32)], val)
                plsc.store_scatter(
                    dlist_v,
                    [jnp.full((LANES,), dslot * DCAP + jnp.minimum(d, DCAP - 1),
                              jnp.int32)],
                    jnp.full((LANES,), cc, jnp.int32), mask=lane0)
                return d + 1

            return lax.fori_loop(0, nb, wb, 0)

        def issue(stage, s, width, sem):
            base = pl.multiple_of(col_lo + s * BLOCK, 128)
            return pltpu.async_copy(
                stage.at[:, pl.ds(0, width)],
                out_hbm.at[:, pl.ds(base, width)], sem)

        def drain(stage, width, sem):
            pltpu.make_async_copy(
                stage.at[:, pl.ds(0, width)],
                out_hbm.at[:, pl.ds(0, width)], sem).wait()

        def pair_body(g, carry):
            da, db = carry

            def one(s, stage, sem, dslot, d):
                @pl.when(g > 0)
                def _():
                    drain(stage, BLOCK, sem)
                    rezero(stage, dslot, d)
                d2 = fill_block(s, stage, dslot)
                issue(stage, s, BLOCK, sem)
                return d2

            da = one(2 * g, stage_a, sem_a, 0, da)
            db = one(2 * g + 1, stage_b, sem_b, 1, db)
            return da, db

        da, db = lax.fori_loop(0, NPAIR, pair_body, (0, 0))

        drain(stage_a, BLOCK, sem_a)
        rezero(stage_a, 0, da)
        fill_block(30, stage_a, 0)

        @pl.when(has_tail)
        def _():
            issue(stage_a, 30, 640, sem_a)
            drain(stage_a, 640, sem_a)

        @pl.when(jnp.logical_not(has_tail))
        def _():
            issue(stage_a, 30, 512, sem_a)
            drain(stage_a, 512, sem_a)

        drain(stage_b, BLOCK, sem_b)

    return run(ids, *planes)


def kernel(risk_data):
    ids = risk_data[:, 16].astype(jnp.int32)
    planes = tuple(risk_data[:, c].astype(jnp.float32) for c in range(1, 16))
    out_t = _scatter_table_t(ids, *planes)
    return out_t.T


# speedup vs baseline: 1.0053x; 1.0053x over previous
import functools

import jax
import jax.numpy as jnp
from jax import lax
from jax.experimental import pallas as pl
from jax.experimental.pallas import tpu as pltpu
from jax.experimental.pallas import tpu_sc as plsc

N_ROWS = 16384
TABLE_ROWS = 1_000_000
BASIC = 16
LANES = 16
NFEAT = 15

NUM_CORES = 2
NUM_SUBCORES = 16
NUM_WORKERS = NUM_CORES * NUM_SUBCORES
TILES_SMALL = 244
BLOCK = 1024
NPAIR = 15
CAP = 1024
CAPC = CAP + LANES
GCHUNK = 128
NCHUNKS = CAP // GCHUNK
DCAP = 64


@jax.jit
def _scatter_table_t(ids, *planes):
    mesh = plsc.VectorSubcoreMesh(core_axis_name="core", subcore_axis_name="subcore")

    @functools.partial(
        pl.kernel,
        out_type=jax.ShapeDtypeStruct((BASIC, TABLE_ROWS), jnp.float32),
        mesh=mesh,
        compiler_params=pltpu.CompilerParams(needs_layout_passes=False,
                                             disable_bounds_checks=True),
        scratch_types=[
            pltpu.VMEM((N_ROWS,), jnp.int32),
            pltpu.VMEM((CAPC,), jnp.int32),
            pltpu.VMEM((CAPC,), jnp.int32),
            pltpu.VMEM((BASIC * CAPC,), jnp.float32),
            pltpu.VMEM((BASIC, BLOCK), jnp.float32),
            pltpu.VMEM((BASIC, BLOCK), jnp.float32),
            pltpu.VMEM((CAPC,), jnp.int32),
            pltpu.VMEM((CAPC,), jnp.int32),
            pltpu.VMEM((2 * DCAP + LANES,), jnp.int32),
            pltpu.SemaphoreType.DMA,
            pltpu.SemaphoreType.DMA,
            pltpu.SemaphoreType.DMA,
        ],
    )
    def run(ids_hbm, *rest):
        plane_hbm = rest[:NFEAT]
        out_hbm = rest[NFEAT]
        (ids_v, rows_l, ids_l, vals_v, stage_a, stage_b, subp_v, subid_v,
         dlist_v, sem_g, sem_a, sem_b) = rest[NFEAT + 1:]

        wid = lax.axis_index("subcore") * NUM_CORES + lax.axis_index("core")
        tile_lo = wid * TILES_SMALL + jnp.minimum(wid, 4)
        col_lo = pl.multiple_of(tile_lo * 128, 128)
        ntiles = jnp.where(wid < 4, TILES_SMALL + 1, TILES_SMALL)
        col_hi = col_lo + ntiles * 128
        mask_hi = jnp.where(wid == NUM_WORKERS - 1, TABLE_ROWS, col_hi)
        has_tail = (wid < 4) | (wid == NUM_WORKERS - 1)
        iota = lax.iota(jnp.int32, LANES)
        lane0 = iota == 0
        zrow = jnp.zeros((LANES,), jnp.float32)
        neg1 = jnp.full((LANES,), -1, jnp.int32)
        c17 = jnp.full((LANES,), 17.0, jnp.float32)

        pltpu.sync_copy(ids_hbm, ids_v)

        @pl.loop(0, CAPC // LANES)
        def _(i):
            rows_l[pl.ds(i * LANES, LANES)] = (iota + i * LANES) * 8
            ids_l[pl.ds(i * LANES, LANES)] = neg1
            vals_v[pl.ds((BASIC - 1) * CAPC + i * LANES, LANES)] = c17

        @pl.loop(0, BLOCK // 4)
        def _(i):
            for u in range(4):
                cc = jnp.full((LANES,), i * 4 + u, jnp.int32)
                plsc.store_scatter(stage_a, [iota, cc], zrow)
                plsc.store_scatter(stage_b, [iota, cc], zrow)

        def scan_body(b, cnt):
            idv = ids_v[pl.ds(b * LANES, LANES)]
            m = (idv >= col_lo) & (idv < mask_hi)
            s = jnp.sum(m.astype(jnp.int32))

            @pl.when(s > 0)
            def _():
                plsc.store_compressed(rows_l.at[pl.ds(cnt, LANES)],
                                      iota + b * LANES, mask=m)
                plsc.store_compressed(ids_l.at[pl.ds(cnt, LANES)], idv, mask=m)

            return jnp.minimum(cnt + s, CAP)

        with jax.named_scope("p1_scan"):
            cnt = lax.fori_loop(0, N_ROWS // LANES, scan_body, 0)

        with jax.named_scope("p2_gather"):
          for k in range(NCHUNKS):
            @pl.when(k * GCHUNK < cnt)
            def _():
                idx = rows_l.at[pl.ds(k * GCHUNK, GCHUNK)]
                copies = [
                    pltpu.async_copy(
                        plane_hbm[r].at[idx],
                        vals_v.at[pl.ds(r * CAPC + k * GCHUNK, GCHUNK)],
                        sem_g)
                    for r in range(NFEAT)
                ]
                for c in copies:
                    c.wait()

        def rezero(stage, dslot, dcnt):
            def few(_):
                def zb(q, _):
                    cc = dlist_v[pl.ds(dslot * DCAP + q, LANES)][0]
                    plsc.store_scatter(stage,
                                       [iota, jnp.full((LANES,), cc, jnp.int32)],
                                       zrow)
                    return 0
                lax.fori_loop(0, dcnt, zb, 0)
                return 0

            def full(_):
                def zb(c, _):
                    plsc.store_scatter(stage,
                                       [iota, jnp.full((LANES,), c, jnp.int32)],
                                       zrow)
                    return 0
                lax.fori_loop(0, BLOCK, zb, 0)
                return 0

            lax.cond(dcnt <= DCAP, few, full, 0)

        def fill_block(s, stage, dslot):
            base = col_lo + s * BLOCK

            def sb(i, sc):
                idv = ids_l[pl.ds(i * LANES, LANES)]
                m = (idv >= base) & (idv < base + BLOCK)
                s2 = jnp.sum(m.astype(jnp.int32))

                @pl.when(s2 > 0)
                def _():
                    plsc.store_compressed(subp_v.at[pl.ds(sc, LANES)],
                                          iota + i * LANES, mask=m)
                    plsc.store_compressed(subid_v.at[pl.ds(sc, LANES)], idv,
                                          mask=m)

                return sc + s2

            nb = lax.fori_loop(0, (cnt + LANES - 1) // LANES, sb, 0)

            def wb(q, d):
                sp = subp_v[pl.ds(q, LANES)][0]
                sid = subid_v[pl.ds(q, LANES)][0]
                cc = sid - base
                val = plsc.load_gather(vals_v,
                                       [iota * CAPC + jnp.full((LANES,), sp,
                                                               jnp.int32)])
                plsc.store_scatter(stage,
                                   [iota, jnp.full((LANES,), cc, jnp.int32)], val)
                plsc.store_scatter(
                    dlist_v,
                    [jnp.full((LANES,), dslot * DCAP + jnp.minimum(d, DCAP - 1),
                              jnp.int32)],
                    jnp.full((LANES,), cc, jnp.int32), mask=lane0)
                return d + 1

            return lax.fori_loop(0, nb, wb, 0)

        def issue(stage, s, width, sem):
            base = pl.multiple_of(col_lo + s * BLOCK, 128)
            return pltpu.async_copy(
                stage.at[:, pl.ds(0, width)],
                out_hbm.at[:, pl.ds(base, width)], sem)

        def drain(stage, width, sem):
            pltpu.make_async_copy(
                stage.at[:, pl.ds(0, width)],
                out_hbm.at[:, pl.ds(0, width)], sem).wait()

        def pair_body(g, carry):
            da, db = carry

            def one(s, stage, sem, dslot, d):
                @pl.when(g > 0)
                def _():
                    drain(stage, BLOCK, sem)
                    rezero(stage, dslot, d)
                d2 = fill_block(s, stage, dslot)
                issue(stage, s, BLOCK, sem)
                return d2

            da = one(2 * g, stage_a, sem_a, 0, da)
            db = one(2 * g + 1, stage_b, sem_b, 1, db)
            return da, db

        with jax.named_scope("p5_blocks"):
            da, db = lax.fori_loop(0, NPAIR, pair_body, (0, 0))

        drain(stage_a, BLOCK, sem_a)
        rezero(stage_a, 0, da)
        fill_block(30, stage_a, 0)

        @pl.when(has_tail)
        def _():
            issue(stage_a, 30, 640, sem_a)
            drain(stage_a, 640, sem_a)

        @pl.when(jnp.logical_not(has_tail))
        def _():
            issue(stage_a, 30, 512, sem_a)
            drain(stage_a, 512, sem_a)

        drain(stage_b, BLOCK, sem_b)

    return run(ids, *planes)


def kernel(risk_data):
    ids = risk_data[:, 16].astype(jnp.int32)
    planes = tuple(risk_data[:, c].astype(jnp.float32) for c in range(1, 16))
    out_t = _scatter_table_t(ids, *planes)
    return out_t.T


# speedup vs baseline: 1.2678x; 1.2611x over previous
import functools

import jax
import jax.numpy as jnp
from jax import lax
from jax.experimental import pallas as pl
from jax.experimental.pallas import tpu as pltpu
from jax.experimental.pallas import tpu_sc as plsc

N_ROWS = 16384
TABLE_ROWS = 1_000_000
BASIC = 16
LANES = 16
NFEAT = 15

NUM_CORES = 2
NUM_SUBCORES = 16
NUM_WORKERS = NUM_CORES * NUM_SUBCORES
TILES_SMALL = 244
BLOCK = 1024
NPAIR = 15
CAP = 1024
CAPC = CAP + LANES
GCHUNK = 128
NCHUNKS = CAP // GCHUNK
DCAP = 64


@jax.jit
def _scatter_table_t(ids, *planes):
    mesh = plsc.VectorSubcoreMesh(core_axis_name="core", subcore_axis_name="subcore")

    @functools.partial(
        pl.kernel,
        out_type=jax.ShapeDtypeStruct((BASIC, TABLE_ROWS), jnp.float32),
        mesh=mesh,
        compiler_params=pltpu.CompilerParams(needs_layout_passes=False,
                                             disable_bounds_checks=True),
        scratch_types=[
            pltpu.VMEM((N_ROWS,), jnp.int32),
            pltpu.VMEM((CAPC,), jnp.int32),
            pltpu.VMEM((CAPC,), jnp.int32),
            pltpu.VMEM((BASIC * CAPC,), jnp.float32),
            pltpu.VMEM((BASIC, BLOCK), jnp.float32),
            pltpu.VMEM((BASIC, BLOCK), jnp.float32),
            pltpu.VMEM((CAPC,), jnp.int32),
            pltpu.VMEM((CAPC,), jnp.int32),
            pltpu.VMEM((2 * DCAP + LANES,), jnp.int32),
            pltpu.SemaphoreType.DMA,
            pltpu.SemaphoreType.DMA,
            pltpu.SemaphoreType.DMA,
        ],
    )
    def run(ids_hbm, *rest):
        plane_hbm = rest[:NFEAT]
        out_hbm = rest[NFEAT]
        (ids_v, rows_l, ids_l, vals_v, stage_a, stage_b, subp_v, subid_v,
         dlist_v, sem_g, sem_a, sem_b) = rest[NFEAT + 1:]

        wid = lax.axis_index("subcore") * NUM_CORES + lax.axis_index("core")
        tile_lo = wid * TILES_SMALL + jnp.minimum(wid, 4)
        col_lo = pl.multiple_of(tile_lo * 128, 128)
        ntiles = jnp.where(wid < 4, TILES_SMALL + 1, TILES_SMALL)
        col_hi = col_lo + ntiles * 128
        mask_hi = jnp.where(wid == NUM_WORKERS - 1, TABLE_ROWS, col_hi)
        has_tail = (wid < 4) | (wid == NUM_WORKERS - 1)
        iota = lax.iota(jnp.int32, LANES)
        lane0 = iota == 0
        zrow = jnp.zeros((LANES,), jnp.float32)
        neg1 = jnp.full((LANES,), -1, jnp.int32)
        c17 = jnp.full((LANES,), 17.0, jnp.float32)

        ids_copy = pltpu.async_copy(ids_hbm, ids_v, sem_g)

        @pl.loop(0, CAPC // LANES)
        def _(i):
            rows_l[pl.ds(i * LANES, LANES)] = (iota + i * LANES) * 8
            ids_l[pl.ds(i * LANES, LANES)] = neg1
            vals_v[pl.ds((BASIC - 1) * CAPC + i * LANES, LANES)] = c17

        @pl.loop(0, BLOCK // LANES)
        def _(j):
            for r in range(BASIC):
                stage_a[r, pl.ds(j * LANES, LANES)] = zrow
                stage_b[r, pl.ds(j * LANES, LANES)] = zrow

        ids_copy.wait()

        def scan_body(b, cnt):
            idv = ids_v[pl.ds(b * LANES, LANES)]
            m = (idv >= col_lo) & (idv < mask_hi)
            s = plsc.all_reduce_population_count(m)[0]
            plsc.store_compressed(rows_l.at[pl.ds(cnt, LANES)],
                                  iota + b * LANES, mask=m)
            plsc.store_compressed(ids_l.at[pl.ds(cnt, LANES)], idv, mask=m)
            return jnp.minimum(cnt + s, CAP)

        with jax.named_scope("p1_scan"):
            cnt = lax.fori_loop(0, N_ROWS // LANES, scan_body, 0)

        with jax.named_scope("p2_gather"):
            copies = [
                pltpu.async_copy(
                    plane_hbm[r].at[rows_l.at[pl.ds(k * GCHUNK, GCHUNK)]],
                    vals_v.at[pl.ds(r * CAPC + k * GCHUNK, GCHUNK)],
                    sem_g)
                for k in range(NCHUNKS)
                for r in range(NFEAT)
            ]
            for c in copies:
                c.wait()

        def rezero(stage, dslot, dcnt):
            def few(_):
                def zb(q, _):
                    cc = dlist_v[pl.ds(dslot * DCAP + q, LANES)][0]
                    plsc.store_scatter(stage,
                                       [iota, jnp.full((LANES,), cc, jnp.int32)],
                                       zrow)
                    return 0
                lax.fori_loop(0, dcnt, zb, 0)
                return 0

            def full(_):
                def zb(j, _):
                    for r in range(BASIC):
                        stage[r, pl.ds(j * LANES, LANES)] = zrow
                    return 0
                lax.fori_loop(0, BLOCK // LANES, zb, 0)
                return 0

            lax.cond(dcnt <= DCAP, few, full, 0)

        def fill_block(s, stage, dslot):
            base = col_lo + s * BLOCK

            def sb(i, sc):
                idv = ids_l[pl.ds(i * LANES, LANES)]
                m = (idv >= base) & (idv < base + BLOCK)
                s2 = plsc.all_reduce_population_count(m)[0]
                plsc.store_compressed(subp_v.at[pl.ds(sc, LANES)],
                                      iota + i * LANES, mask=m)
                plsc.store_compressed(subid_v.at[pl.ds(sc, LANES)], idv,
                                      mask=m)
                return sc + s2

            nb = lax.fori_loop(0, (cnt + LANES - 1) // LANES, sb, 0)

            def wb(q, d):
                sp = subp_v[pl.ds(q, LANES)][0]
                sid = subid_v[pl.ds(q, LANES)][0]
                cc = sid - base
                val = plsc.load_gather(vals_v,
                                       [iota * CAPC + jnp.full((LANES,), sp,
                                                               jnp.int32)])
                plsc.store_scatter(stage,
                                   [iota, jnp.full((LANES,), cc, jnp.int32)], val)
                plsc.store_scatter(
                    dlist_v,
                    [jnp.full((LANES,), dslot * DCAP + jnp.minimum(d, DCAP - 1),
                              jnp.int32)],
                    jnp.full((LANES,), cc, jnp.int32), mask=lane0)
                return d + 1

            return lax.fori_loop(0, nb, wb, 0)

        def issue(stage, s, width, sem):
            base = pl.multiple_of(col_lo + s * BLOCK, 128)
            return pltpu.async_copy(
                stage.at[:, pl.ds(0, width)],
                out_hbm.at[:, pl.ds(base, width)], sem)

        def drain(stage, width, sem):
            pltpu.make_async_copy(
                stage.at[:, pl.ds(0, width)],
                out_hbm.at[:, pl.ds(0, width)], sem).wait()

        def pair_body(g, carry):
            da, db = carry

            def one(s, stage, sem, dslot, d):
                @pl.when(g > 0)
                def _():
                    drain(stage, BLOCK, sem)
                    rezero(stage, dslot, d)
                d2 = fill_block(s, stage, dslot)
                issue(stage, s, BLOCK, sem)
                return d2

            da = one(2 * g, stage_a, sem_a, 0, da)
            db = one(2 * g + 1, stage_b, sem_b, 1, db)
            return da, db

        with jax.named_scope("p5_blocks"):
            da, db = lax.fori_loop(0, NPAIR, pair_body, (0, 0))

        drain(stage_a, BLOCK, sem_a)
        rezero(stage_a, 0, da)
        fill_block(30, stage_a, 0)

        @pl.when(has_tail)
        def _():
            issue(stage_a, 30, 640, sem_a)
            drain(stage_a, 640, sem_a)

        @pl.when(jnp.logical_not(has_tail))
        def _():
            issue(stage_a, 30, 512, sem_a)
            drain(stage_a, 512, sem_a)

        drain(stage_b, BLOCK, sem_b)

    return run(ids, *planes)


def kernel(risk_data):
    ids = risk_data[:, 16].astype(jnp.int32)
    planes = tuple(risk_data[:, c].astype(jnp.float32) for c in range(1, 16))
    out_t = _scatter_table_t(ids, *planes)
    return out_t.T


# speedup vs baseline: 1.4329x; 1.1302x over previous
import functools

import jax
import jax.numpy as jnp
from jax import lax
from jax.experimental import pallas as pl
from jax.experimental.pallas import tpu as pltpu
from jax.experimental.pallas import tpu_sc as plsc

N_ROWS = 16384
TABLE_ROWS = 1_000_000
BASIC = 16
LANES = 16
NFEAT = 15

NUM_CORES = 2
NUM_SUBCORES = 16
NUM_WORKERS = NUM_CORES * NUM_SUBCORES
TILES_SMALL = 244
BLOCK = 1024
NPAIR = 15
CAP = 1024
CAPC = CAP + LANES
GCHUNK = 128
NCHUNKS = CAP // GCHUNK
DCAP = 64


@jax.jit
def _scatter_table_t(ids, *planes):
    mesh = plsc.VectorSubcoreMesh(core_axis_name="core", subcore_axis_name="subcore")

    @functools.partial(
        pl.kernel,
        out_type=jax.ShapeDtypeStruct((BASIC, TABLE_ROWS), jnp.float32),
        mesh=mesh,
        compiler_params=pltpu.CompilerParams(needs_layout_passes=False,
                                             disable_bounds_checks=True),
        scratch_types=[
            pltpu.VMEM((N_ROWS,), jnp.int32),
            pltpu.VMEM((CAPC,), jnp.int32),
            pltpu.VMEM((CAPC,), jnp.int32),
            pltpu.VMEM((BASIC * CAPC,), jnp.float32),
            pltpu.VMEM((BASIC, BLOCK), jnp.float32),
            pltpu.VMEM((BASIC, BLOCK), jnp.float32),
            pltpu.VMEM((CAPC,), jnp.int32),
            pltpu.VMEM((CAPC,), jnp.int32),
            pltpu.VMEM((2 * DCAP + LANES,), jnp.int32),
            pltpu.SemaphoreType.DMA,
            pltpu.SemaphoreType.DMA,
            pltpu.SemaphoreType.DMA,
        ],
    )
    def run(ids_hbm, *rest):
        plane_hbm = rest[:NFEAT]
        out_hbm = rest[NFEAT]
        (ids_v, rows_l, ids_l, vals_v, stage_a, stage_b, subp_v, subid_v,
         dlist_v, sem_g, sem_a, sem_b) = rest[NFEAT + 1:]

        wid = lax.axis_index("subcore") * NUM_CORES + lax.axis_index("core")
        tile_lo = wid * TILES_SMALL + jnp.minimum(wid, 4)
        col_lo = pl.multiple_of(tile_lo * 128, 128)
        ntiles = jnp.where(wid < 4, TILES_SMALL + 1, TILES_SMALL)
        col_hi = col_lo + ntiles * 128
        mask_hi = jnp.where(wid == NUM_WORKERS - 1, TABLE_ROWS, col_hi)
        has_tail = (wid < 4) | (wid == NUM_WORKERS - 1)
        iota = lax.iota(jnp.int32, LANES)
        lane0 = iota == 0
        zrow = jnp.zeros((LANES,), jnp.float32)
        neg1 = jnp.full((LANES,), -1, jnp.int32)
        c17 = jnp.full((LANES,), 17.0, jnp.float32)

        ids_copy = pltpu.async_copy(ids_hbm, ids_v, sem_g)

        @pl.loop(0, CAPC // LANES)
        def _(i):
            rows_l[pl.ds(i * LANES, LANES)] = (iota + i * LANES) * 8
            ids_l[pl.ds(i * LANES, LANES)] = neg1
            vals_v[pl.ds((BASIC - 1) * CAPC + i * LANES, LANES)] = c17

        @pl.loop(0, BLOCK // LANES)
        def _(j):
            for r in range(BASIC):
                stage_a[r, pl.ds(j * LANES, LANES)] = zrow
                stage_b[r, pl.ds(j * LANES, LANES)] = zrow

        ids_copy.wait()

        def scan_body(b, cnt):
            idv0 = ids_v[pl.ds(2 * b * LANES, LANES)]
            idv1 = ids_v[pl.ds((2 * b + 1) * LANES, LANES)]
            m0 = (idv0 >= col_lo) & (idv0 < mask_hi)
            m1 = (idv1 >= col_lo) & (idv1 < mask_hi)
            s0 = plsc.all_reduce_population_count(m0)[0]
            s1 = plsc.all_reduce_population_count(m1)[0]
            plsc.store_compressed(rows_l.at[pl.ds(cnt, LANES)],
                                  iota + 2 * b * LANES, mask=m0)
            plsc.store_compressed(ids_l.at[pl.ds(cnt, LANES)], idv0, mask=m0)
            c1 = jnp.minimum(cnt + s0, CAP)
            plsc.store_compressed(rows_l.at[pl.ds(c1, LANES)],
                                  iota + (2 * b + 1) * LANES, mask=m1)
            plsc.store_compressed(ids_l.at[pl.ds(c1, LANES)], idv1, mask=m1)
            return jnp.minimum(c1 + s1, CAP)

        with jax.named_scope("p1_scan"):
            cnt = lax.fori_loop(0, N_ROWS // LANES // 2, scan_body, 0)

        with jax.named_scope("p2_gather"):
          for k in range(NCHUNKS):
            @pl.when(k * GCHUNK < cnt)
            def _():
                idx = rows_l.at[pl.ds(k * GCHUNK, GCHUNK)]
                copies = [
                    pltpu.async_copy(
                        plane_hbm[r].at[idx],
                        vals_v.at[pl.ds(r * CAPC + k * GCHUNK, GCHUNK)],
                        sem_g)
                    for r in range(NFEAT)
                ]
                for c in copies:
                    c.wait()

        def rezero(stage, dslot, dcnt):
            def few(_):
                def zb(q, _):
                    cc = dlist_v[pl.ds(dslot * DCAP + q, LANES)][0]
                    plsc.store_scatter(stage,
                                       [iota, jnp.full((LANES,), cc, jnp.int32)],
                                       zrow)
                    return 0
                lax.fori_loop(0, dcnt, zb, 0)
                return 0

            def full(_):
                def zb(j, _):
                    for r in range(BASIC):
                        stage[r, pl.ds(j * LANES, LANES)] = zrow
                    return 0
                lax.fori_loop(0, BLOCK // LANES, zb, 0)
                return 0

            lax.cond(dcnt <= DCAP, few, full, 0)

        def fill_block(s, stage, dslot):
            base = col_lo + s * BLOCK

            def sb(i, sc):
                idv = ids_l[pl.ds(i * LANES, LANES)]
                m = (idv >= base) & (idv < base + BLOCK)
                s2 = plsc.all_reduce_population_count(m)[0]
                plsc.store_compressed(subp_v.at[pl.ds(sc, LANES)],
                                      iota + i * LANES, mask=m)
                plsc.store_compressed(subid_v.at[pl.ds(sc, LANES)], idv,
                                      mask=m)
                return sc + s2

            nb = lax.fori_loop(0, (cnt + LANES - 1) // LANES, sb, 0)

            def wb(q, d):
                sp = subp_v[pl.ds(q, LANES)][0]
                sid = subid_v[pl.ds(q, LANES)][0]
                cc = sid - base
                val = plsc.load_gather(vals_v,
                                       [iota * CAPC + jnp.full((LANES,), sp,
                                                               jnp.int32)])
                plsc.store_scatter(stage,
                                   [iota, jnp.full((LANES,), cc, jnp.int32)], val)
                plsc.store_scatter(
                    dlist_v,
                    [jnp.full((LANES,), dslot * DCAP + jnp.minimum(d, DCAP - 1),
                              jnp.int32)],
                    jnp.full((LANES,), cc, jnp.int32), mask=lane0)
                return d + 1

            return lax.fori_loop(0, nb, wb, 0)

        def issue(stage, s, width, sem):
            base = pl.multiple_of(col_lo + s * BLOCK, 128)
            return pltpu.async_copy(
                stage.at[:, pl.ds(0, width)],
                out_hbm.at[:, pl.ds(base, width)], sem)

        def drain(stage, width, sem):
            pltpu.make_async_copy(
                stage.at[:, pl.ds(0, width)],
                out_hbm.at[:, pl.ds(0, width)], sem).wait()

        def pair_body(g, carry):
            da, db = carry

            def one(s, stage, sem, dslot, d):
                @pl.when(g > 0)
                def _():
                    drain(stage, BLOCK, sem)
                    rezero(stage, dslot, d)
                d2 = fill_block(s, stage, dslot)
                issue(stage, s, BLOCK, sem)
                return d2

            da = one(2 * g, stage_a, sem_a, 0, da)
            db = one(2 * g + 1, stage_b, sem_b, 1, db)
            return da, db

        with jax.named_scope("p5_blocks"):
            da, db = lax.fori_loop(0, NPAIR, pair_body, (0, 0))

        drain(stage_a, BLOCK, sem_a)
        rezero(stage_a, 0, da)
        fill_block(30, stage_a, 0)

        @pl.when(has_tail)
        def _():
            issue(stage_a, 30, 640, sem_a)
            drain(stage_a, 640, sem_a)

        @pl.when(jnp.logical_not(has_tail))
        def _():
            issue(stage_a, 30, 512, sem_a)
            drain(stage_a, 512, sem_a)

        drain(stage_b, BLOCK, sem_b)

    return run(ids, *planes)


def kernel(risk_data):
    ids = risk_data[:, 16].astype(jnp.int32)
    planes = tuple(risk_data[:, c].astype(jnp.float32) for c in range(1, 16))
    out_t = _scatter_table_t(ids, *planes)
    return out_t.T


# speedup vs baseline: 1.5022x; 1.0483x over previous
import functools

import jax
import jax.numpy as jnp
from jax import lax
from jax.experimental import pallas as pl
from jax.experimental.pallas import tpu as pltpu
from jax.experimental.pallas import tpu_sc as plsc

N_ROWS = 16384
TABLE_ROWS = 1_000_000
BASIC = 16
LANES = 16
NFEAT = 15

NUM_CORES = 2
NUM_SUBCORES = 16
NUM_WORKERS = NUM_CORES * NUM_SUBCORES
TILES_SMALL = 244
BLOCK = 1024
NPAIR = 15
CAP = 1024
CAPC = CAP + LANES
GCHUNK = 128
NCHUNKS = CAP // GCHUNK
DCAP = 64


@jax.jit
def _scatter_table_t(ids, *planes):
    mesh = plsc.VectorSubcoreMesh(core_axis_name="core", subcore_axis_name="subcore")

    @functools.partial(
        pl.kernel,
        out_type=jax.ShapeDtypeStruct((BASIC, TABLE_ROWS), jnp.float32),
        mesh=mesh,
        compiler_params=pltpu.CompilerParams(needs_layout_passes=False,
                                             disable_bounds_checks=True),
        scratch_types=[
            pltpu.VMEM((N_ROWS,), jnp.int32),
            pltpu.VMEM((CAPC,), jnp.int32),
            pltpu.VMEM((CAPC,), jnp.int32),
            pltpu.VMEM((BASIC * CAPC,), jnp.float32),
            pltpu.VMEM((BASIC, BLOCK), jnp.float32),
            pltpu.VMEM((BASIC, BLOCK), jnp.float32),
            pltpu.VMEM((CAPC,), jnp.int32),
            pltpu.VMEM((CAPC,), jnp.int32),
            pltpu.VMEM((2 * DCAP + LANES,), jnp.int32),
            pltpu.SemaphoreType.DMA,
            pltpu.SemaphoreType.DMA,
            pltpu.SemaphoreType.DMA,
        ],
    )
    def run(ids_hbm, *rest):
        plane_hbm = rest[:NFEAT]
        out_hbm = rest[NFEAT]
        (ids_v, rows_l, ids_l, vals_v, stage_a, stage_b, subp_v, subid_v,
         dlist_v, sem_g, sem_a, sem_b) = rest[NFEAT + 1:]

        wid = lax.axis_index("subcore") * NUM_CORES + lax.axis_index("core")
        tile_lo = wid * TILES_SMALL + jnp.minimum(wid, 4)
        col_lo = pl.multiple_of(tile_lo * 128, 128)
        ntiles = jnp.where(wid < 4, TILES_SMALL + 1, TILES_SMALL)
        col_hi = col_lo + ntiles * 128
        mask_hi = jnp.where(wid == NUM_WORKERS - 1, TABLE_ROWS, col_hi)
        has_tail = (wid < 4) | (wid == NUM_WORKERS - 1)
        iota = lax.iota(jnp.int32, LANES)
        lane0 = iota == 0
        zrow = jnp.zeros((LANES,), jnp.float32)
        neg1 = jnp.full((LANES,), -1, jnp.int32)
        c17 = jnp.full((LANES,), 17.0, jnp.float32)

        ids_copy = pltpu.async_copy(ids_hbm, ids_v, sem_g)

        @pl.loop(0, CAPC // LANES)
        def _(i):
            rows_l[pl.ds(i * LANES, LANES)] = (iota + i * LANES) * 8
            ids_l[pl.ds(i * LANES, LANES)] = neg1
            vals_v[pl.ds((BASIC - 1) * CAPC + i * LANES, LANES)] = c17

        @pl.loop(0, BLOCK // LANES)
        def _(j):
            for r in range(BASIC):
                stage_a[r, pl.ds(j * LANES, LANES)] = zrow
                stage_b[r, pl.ds(j * LANES, LANES)] = zrow

        ids_copy.wait()

        def scan_body(b, cnt):
            idv0 = ids_v[pl.ds(2 * b * LANES, LANES)]
            idv1 = ids_v[pl.ds((2 * b + 1) * LANES, LANES)]
            m0 = (idv0 >= col_lo) & (idv0 < mask_hi)
            m1 = (idv1 >= col_lo) & (idv1 < mask_hi)
            s0 = plsc.all_reduce_population_count(m0)[0]
            s1 = plsc.all_reduce_population_count(m1)[0]
            plsc.store_compressed(rows_l.at[pl.ds(cnt, LANES)],
                                  iota + 2 * b * LANES, mask=m0)
            plsc.store_compressed(ids_l.at[pl.ds(cnt, LANES)], idv0, mask=m0)
            c1 = jnp.minimum(cnt + s0, CAP)
            plsc.store_compressed(rows_l.at[pl.ds(c1, LANES)],
                                  iota + (2 * b + 1) * LANES, mask=m1)
            plsc.store_compressed(ids_l.at[pl.ds(c1, LANES)], idv1, mask=m1)
            return jnp.minimum(c1 + s1, CAP)

        with jax.named_scope("p1_scan"):
            cnt = lax.fori_loop(0, N_ROWS // LANES // 2, scan_body, 0)

        with jax.named_scope("p2_gather"):
            for k in range(NCHUNKS):
                @pl.when(k * GCHUNK < cnt)
                def _():
                    idx = rows_l.at[pl.ds(k * GCHUNK, GCHUNK)]
                    for r in range(NFEAT):
                        pltpu.async_copy(
                            plane_hbm[r].at[idx],
                            vals_v.at[pl.ds(r * CAPC + k * GCHUNK, GCHUNK)],
                            sem_g)

            nact = (cnt + GCHUNK - 1) // GCHUNK

            def drain_g(i, _):
                pltpu.make_async_copy(
                    plane_hbm[0].at[rows_l.at[pl.ds(0, GCHUNK)]],
                    vals_v.at[pl.ds(0, GCHUNK)], sem_g).wait()
                return 0

            lax.fori_loop(0, nact * NFEAT, drain_g, 0)

        def rezero(stage, dslot, dcnt):
            def few(_):
                def zb(q, _):
                    cc = dlist_v[pl.ds(dslot * DCAP + q, LANES)][0]
                    plsc.store_scatter(stage,
                                       [iota, jnp.full((LANES,), cc, jnp.int32)],
                                       zrow)
                    return 0
                lax.fori_loop(0, dcnt, zb, 0)
                return 0

            def full(_):
                def zb(j, _):
                    for r in range(BASIC):
                        stage[r, pl.ds(j * LANES, LANES)] = zrow
                    return 0
                lax.fori_loop(0, BLOCK // LANES, zb, 0)
                return 0

            lax.cond(dcnt <= DCAP, few, full, 0)

        def fill_block(s, stage, dslot):
            base = col_lo + s * BLOCK

            def sb(i, sc):
                idv = ids_l[pl.ds(i * LANES, LANES)]
                m = (idv >= base) & (idv < base + BLOCK)
                s2 = plsc.all_reduce_population_count(m)[0]
                plsc.store_compressed(subp_v.at[pl.ds(sc, LANES)],
                                      iota + i * LANES, mask=m)
                plsc.store_compressed(subid_v.at[pl.ds(sc, LANES)], idv,
                                      mask=m)
                return sc + s2

            nb = lax.fori_loop(0, (cnt + LANES - 1) // LANES, sb, 0)

            def wb(q, d):
                sp = subp_v[pl.ds(q, LANES)][0]
                sid = subid_v[pl.ds(q, LANES)][0]
                cc = sid - base
                val = plsc.load_gather(vals_v,
                                       [iota * CAPC + jnp.full((LANES,), sp,
                                                               jnp.int32)])
                plsc.store_scatter(stage,
                                   [iota, jnp.full((LANES,), cc, jnp.int32)], val)
                plsc.store_scatter(
                    dlist_v,
                    [jnp.full((LANES,), dslot * DCAP + jnp.minimum(d, DCAP - 1),
                              jnp.int32)],
                    jnp.full((LANES,), cc, jnp.int32), mask=lane0)
                return d + 1

            return lax.fori_loop(0, nb, wb, 0)

        def issue(stage, s, width, sem):
            base = pl.multiple_of(col_lo + s * BLOCK, 128)
            return pltpu.async_copy(
                stage.at[:, pl.ds(0, width)],
                out_hbm.at[:, pl.ds(base, width)], sem)

        def drain(stage, width, sem):
            pltpu.make_async_copy(
                stage.at[:, pl.ds(0, width)],
                out_hbm.at[:, pl.ds(0, width)], sem).wait()

        def pair_body(g, carry):
            da, db = carry

            def one(s, stage, sem, dslot, d):
                @pl.when(g > 0)
                def _():
                    drain(stage, BLOCK, sem)
                    rezero(stage, dslot, d)
                d2 = fill_block(s, stage, dslot)
                issue(stage, s, BLOCK, sem)
                return d2

            da = one(2 * g, stage_a, sem_a, 0, da)
            db = one(2 * g + 1, stage_b, sem_b, 1, db)
            return da, db

        with jax.named_scope("p5_blocks"):
            da, db = lax.fori_loop(0, NPAIR, pair_body, (0, 0))

        drain(stage_a, BLOCK, sem_a)
        rezero(stage_a, 0, da)
        fill_block(30, stage_a, 0)

        @pl.when(has_tail)
        def _():
            issue(stage_a, 30, 640, sem_a)
            drain(stage_a, 640, sem_a)

        @pl.when(jnp.logical_not(has_tail))
        def _():
            issue(stage_a, 30, 512, sem_a)
            drain(stage_a, 512, sem_a)

        drain(stage_b, BLOCK, sem_b)

    return run(ids, *planes)


def kernel(risk_data):
    ids = risk_data[:, 16].astype(jnp.int32)
    planes = tuple(risk_data[:, c].astype(jnp.float32) for c in range(1, 16))
    out_t = _scatter_table_t(ids, *planes)
    return out_t.T
